# baseline (speedup 1.0000x reference)
import jax
import jax.numpy as jnp
from jax import lax
from jax.experimental import pallas as pl
from jax.experimental.pallas import tpu as pltpu
from jax.experimental.pallas import tpu_sc as plsc

_GRID_R, _GRID_C = 28, 28
_CELLS = _GRID_R * _GRID_C
_BOX_NUM = 3
_CLASS_NUM = 80
_F = 5 * _BOX_NUM + _CLASS_NUM
_B = 64
_N = _B * _CELLS
_RP = 32

_TC_STEPS = 7
_RB = 2
_HR = _GRID_R // 2

_NC, _NS, _L = 2, 16, 16
_NW = _NC * _NS
_B_W = _B // _NW
_CB = 4


def _tc_body(p_ref, gt_ref, loc_ref, out_ref, acc_ref):
    g = pl.program_id(0)

    @pl.when(g == 0)
    def _init():
        acc_ref[0] = 0.0
        acc_ref[1] = 0.0

    p = p_ref[...]
    gt = gt_ref[...]
    loc_ref[..., 0:16] = p[..., 0:16]
    loc_ref[..., 16:32] = gt[..., 0:16]

    pos = (gt[..., 0:1] > 0.0).astype(jnp.float32)
    d = p[..., 15:_F] - gt[..., 15:_F]
    acc_ref[0] = acc_ref[0] + jnp.sum(d * d * pos)
    acc_ref[1] = acc_ref[1] + jnp.sum(pos)

    @pl.when(g == _TC_STEPS - 1)
    def _fin():
        out_ref[0] = acc_ref[0]
        out_ref[1] = acc_ref[1]


def _tc_class(pT_h, gtT_h):
    return pl.pallas_call(
        _tc_body,
        grid=(_TC_STEPS,),
        in_specs=[
            pl.BlockSpec((_RB, _GRID_C, _B, _F), lambda g: (g, 0, 0, 0)),
            pl.BlockSpec((_RB, _GRID_C, _B, _F), lambda g: (g, 0, 0, 0)),
        ],
        out_specs=[
            pl.BlockSpec((_RB, _GRID_C, _B, 128), lambda g: (g, 0, 0, 0)),
            pl.BlockSpec(memory_space=pltpu.SMEM),
        ],
        out_shape=[
            jax.ShapeDtypeStruct((_HR, _GRID_C, _B, 128), jnp.float32),
            jax.ShapeDtypeStruct((2,), jnp.float32),
        ],
        scratch_shapes=[pltpu.SMEM((2,), jnp.float32)],
    )(pT_h, gtT_h)


def _loc_math(pc, gc, ii, jj):
    c0, c1, c2 = pc[0], pc[5], pc[10]
    best = (
        (c0 >= c1) & (c0 >= c2),
        (c1 > c0) & (c1 >= c2),
        (c2 > c0) & (c2 > c1),
    )
    pos = gc[0] > 0.0

    zero = jnp.zeros_like(c0)
    xy_p = zero
    wh_p = zero
    pc_p = zero
    nc_p = zero
    for k in range(_BOX_NUM):
        ck = (c0, c1, c2)[k]
        m = jnp.where(pos & best[k], 1.0, 0.0)
        px, py, pw, ph = pc[5 * k + 1], pc[5 * k + 2], pc[5 * k + 3], pc[5 * k + 4]
        gx, gy, gw, gh = gc[5 * k + 1], gc[5 * k + 2], gc[5 * k + 3], gc[5 * k + 4]

        dx = px - (gx * float(_GRID_C) - jj)
        dy = py - (gy * float(_GRID_R) - ii)
        dw = pw - gw
        dh = ph - gh
        xy_p = xy_p + m * (dx * dx + dy * dy)
        wh_p = wh_p + m * (dw * dw + dh * dh)

        pxg = (px + jj) / float(_GRID_C)
        pyg = (py + ii) / float(_GRID_R)
        ax1 = pxg - pw * 0.5
        ax2 = pxg + pw * 0.5
        ay1 = pyg - ph * 0.5
        ay2 = pyg + ph * 0.5
        bx1 = gx - gw * 0.5
        bx2 = gx + gw * 0.5
        by1 = gy - gh * 0.5
        by2 = gy + gh * 0.5
        iw = jnp.maximum(jnp.minimum(ax2, bx2) - jnp.maximum(ax1, bx1), 0.0)
        ih = jnp.maximum(jnp.minimum(ay2, by2) - jnp.maximum(ay1, by1), 0.0)
        inter = iw * ih
        area_a = jnp.maximum(pw, 0.0) * jnp.maximum(ph, 0.0)
        area_b = jnp.maximum(gw, 0.0) * jnp.maximum(gh, 0.0)
        iou = inter / (area_a + area_b - inter + 1e-10)

        dc = ck - iou
        pc_p = pc_p + m * dc * dc
        nc_p = nc_p + (1.0 - m) * ck * ck
    return xy_p, wh_p, pc_p, nc_p


def _sc_body(r_off, loc_hbm, o_hbm, lv0, lv1, ov, sem0, sem1, sem_o):
    wid = lax.axis_index("s") * _NC + lax.axis_index("c")

    lane = lax.iota(jnp.int32, _L)
    z = jnp.zeros((_L,), jnp.float32)
    bufs = (lv0, lv1)
    sems = (sem0, sem1)
    n_chunks = (_GRID_C // 2) // _CB
    row = lax.shift_right_logical(wid, 1)
    chalf = lax.bitwise_and(wid, 1)
    c_base = chalf * (_GRID_C // 2)

    if True:
        def issue(ci):
            return pltpu.async_copy(
                loc_hbm.at[row, pl.ds(c_base + ci * _CB, _CB), :, :],
                bufs[ci % 2], sems[ci % 2])

        ii_f = (jnp.full((_L,), row, jnp.int32).astype(jnp.float32)
                + float(r_off))

        def compute(ci, buf, carry):
            def cl_loop(c_l, carry1):
                jj_f = (jnp.full((_L,), c_base + ci * _CB, jnp.int32)
                        .astype(jnp.float32) + c_l.astype(jnp.float32))
                cl_v = jnp.full((_L,), c_l, jnp.int32)

                def bh_loop(bh, carry2):
                    xy_a, wh_a, pc_a, nc_a = carry2
                    b_idx = bh * _L + lane
                    pc = [plsc.load_gather(
                              buf, [cl_v, b_idx, jnp.full((_L,), j, jnp.int32)])
                          for j in range(15)]
                    gc = [plsc.load_gather(
                              buf, [cl_v, b_idx,
                                    jnp.full((_L,), _L + j, jnp.int32)])
                          for j in range(15)]
                    xy_p, wh_p, pc_p, nc_p = _loc_math(pc, gc, ii_f, jj_f)
                    return (xy_a + xy_p, wh_a + wh_p,
                            pc_a + pc_p, nc_a + nc_p)

                return lax.fori_loop(0, _B // _L, bh_loop, carry1)

            return lax.fori_loop(0, _CB, cl_loop, carry)

        acc = (z, z, z, z)
        cps = {0: issue(0)}
        for ci in range(n_chunks):
            if ci + 1 < n_chunks:
                cps[ci + 1] = issue(ci + 1)
            cps[ci].wait()
            acc = compute(ci, bufs[ci % 2], acc)

        xy_a, wh_a, pc_a, nc_a = acc
        ov[0, pl.ds(0, _L)] = xy_a
        ov[0, pl.ds(_L, _L)] = wh_a
        ov[0, pl.ds(2 * _L, _L)] = pc_a
        ov[0, pl.ds(3 * _L, _L)] = nc_a
        pltpu.async_copy(ov, o_hbm.at[wid], sem_o).wait()


def _sc_loc(loc, r_off):
    mesh = plsc.VectorSubcoreMesh(core_axis_name="c", subcore_axis_name="s")
    import functools as _ft
    f = pl.kernel(
        _ft.partial(_sc_body, r_off),
        out_type=jax.ShapeDtypeStruct((_NW, 1, 4 * _L), jnp.float32),
        mesh=mesh,
        compiler_params=pltpu.CompilerParams(needs_layout_passes=False),
        scratch_types=[
            pltpu.VMEM((_CB, _B, 128), jnp.float32),
            pltpu.VMEM((_CB, _B, 128), jnp.float32),
            pltpu.VMEM((1, 4 * _L), jnp.float32),
            pltpu.SemaphoreType.DMA,
            pltpu.SemaphoreType.DMA,
            pltpu.SemaphoreType.DMA,
        ],
    )
    return f(loc)


def _combine_body(sc1_ref, sc2_ref, tc1_ref, tc2_ref, out_ref):
    xy_sum = jnp.sum(sc1_ref[:, 0, 0:_L]) + jnp.sum(sc2_ref[:, 0, 0:_L])
    wh_sum = (jnp.sum(sc1_ref[:, 0, _L:2 * _L])
              + jnp.sum(sc2_ref[:, 0, _L:2 * _L]))
    pc_sum = (jnp.sum(sc1_ref[:, 0, 2 * _L:3 * _L])
              + jnp.sum(sc2_ref[:, 0, 2 * _L:3 * _L]))
    nc_sum = (jnp.sum(sc1_ref[:, 0, 3 * _L:4 * _L])
              + jnp.sum(sc2_ref[:, 0, 3 * _L:4 * _L]))
    class_sum = tc1_ref[0] + tc2_ref[0]
    npos = tc1_ref[1] + tc2_ref[1]
    class_loss = class_sum / jnp.maximum(float(_CLASS_NUM) * npos, 1.0)
    xy_loss = xy_sum / jnp.maximum(2.0 * npos, 1.0)
    wh_loss = wh_sum / jnp.maximum(2.0 * npos, 1.0)
    pos_conf = pc_sum / jnp.maximum(npos, 1.0)
    neg_conf = nc_sum / jnp.maximum(float(_BOX_NUM * _N) - npos, 1.0)
    out_ref[0] = (class_loss + 2.0 * pos_conf + 0.5 * neg_conf
                  + 5.0 * xy_loss + 5.0 * wh_loss)
    out_ref[1] = class_loss
    out_ref[2] = xy_loss
    out_ref[3] = wh_loss
    out_ref[4] = pos_conf
    out_ref[5] = neg_conf


def _combine(sc1, sc2, tc1, tc2):
    return pl.pallas_call(
        _combine_body,
        in_specs=[
            pl.BlockSpec(memory_space=pltpu.VMEM),
            pl.BlockSpec(memory_space=pltpu.VMEM),
            pl.BlockSpec(memory_space=pltpu.SMEM),
            pl.BlockSpec(memory_space=pltpu.SMEM),
        ],
        out_specs=pl.BlockSpec(memory_space=pltpu.SMEM),
        out_shape=jax.ShapeDtypeStruct((6,), jnp.float32),
    )(sc1, sc2, tc1, tc2)


@jax.jit
def _yolo_loss(p, gt):
    pT = jnp.transpose(p, (1, 2, 0, 3))
    gtT = jnp.transpose(gt, (1, 2, 0, 3))
    loc1, tc1 = _tc_class(pT[0:_HR], gtT[0:_HR])
    sc1 = _sc_loc(loc1, 0)
    loc2, tc2 = _tc_class(pT[_HR:_GRID_R], gtT[_HR:_GRID_R])
    sc2 = _sc_loc(loc2, _HR)
    out = _combine(sc1, sc2, tc1, tc2)
    return (out[0], out[1], out[2], out[3], out[4], out[5])


def kernel(p, gt):
    return _yolo_loss(p, gt)


# speedup vs baseline: 1.3620x; 1.3620x over previous
import jax
import jax.numpy as jnp
from jax import lax
from jax.experimental import pallas as pl
from jax.experimental.pallas import tpu as pltpu
from jax.experimental.pallas import tpu_sc as plsc

_GRID_R, _GRID_C = 28, 28
_CELLS = _GRID_R * _GRID_C
_BOX_NUM = 3
_CLASS_NUM = 80
_F = 5 * _BOX_NUM + _CLASS_NUM
_B = 64
_N = _B * _CELLS
_RP = 32

_TC_STEPS = 14
_RB = _GRID_R // _TC_STEPS

_NC, _NS, _L = 2, 16, 16
_NW = _NC * _NS
_B_W = _B // _NW
_CB = 4


def _tc_body(p_ref, gt_ref, loc_ref, out_ref, acc_ref):
    g = pl.program_id(0)

    @pl.when(g == 0)
    def _init():
        acc_ref[0] = 0.0
        acc_ref[1] = 0.0

    p = p_ref[...]
    gt = gt_ref[...]
    loc_ref[..., 0:16] = p[..., 0:16]
    loc_ref[..., 16:32] = gt[..., 0:16]

    pos = (gt[..., 0:1] > 0.0).astype(jnp.float32)
    d = p[..., 15:_F] - gt[..., 15:_F]
    acc_ref[0] = acc_ref[0] + jnp.sum(d * d * pos)
    acc_ref[1] = acc_ref[1] + jnp.sum(pos)

    @pl.when(g == _TC_STEPS - 1)
    def _fin():
        out_ref[0] = acc_ref[0]
        out_ref[1] = acc_ref[1]


def _tc_class(pT, gtT):
    return pl.pallas_call(
        _tc_body,
        grid=(_TC_STEPS,),
        in_specs=[
            pl.BlockSpec((_RB, _GRID_C, _B, _F), lambda g: (g, 0, 0, 0)),
            pl.BlockSpec((_RB, _GRID_C, _B, _F), lambda g: (g, 0, 0, 0)),
        ],
        out_specs=[
            pl.BlockSpec((_RB, _GRID_C, _B, 128), lambda g: (g, 0, 0, 0)),
            pl.BlockSpec(memory_space=pltpu.SMEM),
        ],
        out_shape=[
            jax.ShapeDtypeStruct((_GRID_R, _GRID_C, _B, 128), jnp.float32),
            jax.ShapeDtypeStruct((2,), jnp.float32),
        ],
        scratch_shapes=[pltpu.SMEM((2,), jnp.float32)],
    )(pT, gtT)


def _loc_math(pc, gc, ii, jj):
    c0, c1, c2 = pc[0], pc[5], pc[10]
    best = (
        (c0 >= c1) & (c0 >= c2),
        (c1 > c0) & (c1 >= c2),
        (c2 > c0) & (c2 > c1),
    )
    pos = gc[0] > 0.0

    zero = jnp.zeros_like(c0)
    xy_p = zero
    wh_p = zero
    pc_p = zero
    nc_p = zero
    for k in range(_BOX_NUM):
        ck = (c0, c1, c2)[k]
        m = jnp.where(pos & best[k], 1.0, 0.0)
        px, py, pw, ph = pc[5 * k + 1], pc[5 * k + 2], pc[5 * k + 3], pc[5 * k + 4]
        gx, gy, gw, gh = gc[5 * k + 1], gc[5 * k + 2], gc[5 * k + 3], gc[5 * k + 4]

        dx = px - (gx * float(_GRID_C) - jj)
        dy = py - (gy * float(_GRID_R) - ii)
        dw = pw - gw
        dh = ph - gh
        xy_p = xy_p + m * (dx * dx + dy * dy)
        wh_p = wh_p + m * (dw * dw + dh * dh)

        pxg = (px + jj) / float(_GRID_C)
        pyg = (py + ii) / float(_GRID_R)
        ax1 = pxg - pw * 0.5
        ax2 = pxg + pw * 0.5
        ay1 = pyg - ph * 0.5
        ay2 = pyg + ph * 0.5
        bx1 = gx - gw * 0.5
        bx2 = gx + gw * 0.5
        by1 = gy - gh * 0.5
        by2 = gy + gh * 0.5
        iw = jnp.maximum(jnp.minimum(ax2, bx2) - jnp.maximum(ax1, bx1), 0.0)
        ih = jnp.maximum(jnp.minimum(ay2, by2) - jnp.maximum(ay1, by1), 0.0)
        inter = iw * ih
        area_a = jnp.maximum(pw, 0.0) * jnp.maximum(ph, 0.0)
        area_b = jnp.maximum(gw, 0.0) * jnp.maximum(gh, 0.0)
        iou = inter / (area_a + area_b - inter + 1e-10)

        dc = ck - iou
        pc_p = pc_p + m * dc * dc
        nc_p = nc_p + (1.0 - m) * ck * ck
    return xy_p, wh_p, pc_p, nc_p


def _sc_body(loc_hbm, o_hbm, lv0, lv1, ov, sem0, sem1, sem_o):
    wid = lax.axis_index("s") * _NC + lax.axis_index("c")

    lane = lax.iota(jnp.int32, _L)
    z = jnp.zeros((_L,), jnp.float32)
    bufs = (lv0, lv1)
    sems = (sem0, sem1)
    n_chunks = _GRID_C // _CB

    @pl.when(wid < _GRID_R)
    def _work():
        def issue(ci):
            return pltpu.async_copy(
                loc_hbm.at[wid, pl.ds(ci * _CB, _CB), :, :],
                bufs[ci % 2], sems[ci % 2])

        ii_f = jnp.full((_L,), wid, jnp.int32).astype(jnp.float32)

        def compute(ci, buf, carry):
            def cl_loop(c_l, carry1):
                jj_f = jnp.full((_L,), ci * _CB, jnp.int32).astype(
                    jnp.float32) + c_l.astype(jnp.float32)
                cl_v = jnp.full((_L,), c_l, jnp.int32)

                def bh_loop(bh, carry2):
                    xy_a, wh_a, pc_a, nc_a = carry2
                    b_idx = bh * _L + lane
                    pc = [plsc.load_gather(
                              buf, [cl_v, b_idx, jnp.full((_L,), j, jnp.int32)])
                          for j in range(15)]
                    gc = [plsc.load_gather(
                              buf, [cl_v, b_idx,
                                    jnp.full((_L,), _L + j, jnp.int32)])
                          for j in range(15)]
                    xy_p, wh_p, pc_p, nc_p = _loc_math(pc, gc, ii_f, jj_f)
                    return (xy_a + xy_p, wh_a + wh_p,
                            pc_a + pc_p, nc_a + nc_p)

                return lax.fori_loop(0, _B // _L, bh_loop, carry1)

            return lax.fori_loop(0, _CB, cl_loop, carry)

        acc = (z, z, z, z)
        cps = {0: issue(0)}
        for ci in range(n_chunks):
            if ci + 1 < n_chunks:
                cps[ci + 1] = issue(ci + 1)
            cps[ci].wait()
            acc = compute(ci, bufs[ci % 2], acc)

        xy_a, wh_a, pc_a, nc_a = acc
        ov[0, pl.ds(0, _L)] = xy_a
        ov[0, pl.ds(_L, _L)] = wh_a
        ov[0, pl.ds(2 * _L, _L)] = pc_a
        ov[0, pl.ds(3 * _L, _L)] = nc_a
        pltpu.async_copy(ov, o_hbm.at[wid], sem_o).wait()


def _sc_loc(loc):
    mesh = plsc.VectorSubcoreMesh(core_axis_name="c", subcore_axis_name="s")
    f = pl.kernel(
        _sc_body,
        out_type=jax.ShapeDtypeStruct((_NW, 1, 4 * _L), jnp.float32),
        mesh=mesh,
        compiler_params=pltpu.CompilerParams(needs_layout_passes=False),
        scratch_types=[
            pltpu.VMEM((_CB, _B, 128), jnp.float32),
            pltpu.VMEM((_CB, _B, 128), jnp.float32),
            pltpu.VMEM((1, 4 * _L), jnp.float32),
            pltpu.SemaphoreType.DMA,
            pltpu.SemaphoreType.DMA,
            pltpu.SemaphoreType.DMA,
        ],
    )
    return f(loc)


def _combine_body(sc_ref, tc_ref, out_ref):
    xy_sum = jnp.sum(sc_ref[0:_GRID_R, 0, 0:_L])
    wh_sum = jnp.sum(sc_ref[0:_GRID_R, 0, _L:2 * _L])
    pc_sum = jnp.sum(sc_ref[0:_GRID_R, 0, 2 * _L:3 * _L])
    nc_sum = jnp.sum(sc_ref[0:_GRID_R, 0, 3 * _L:4 * _L])
    class_sum = tc_ref[0]
    npos = tc_ref[1]
    class_loss = class_sum / jnp.maximum(float(_CLASS_NUM) * npos, 1.0)
    xy_loss = xy_sum / jnp.maximum(2.0 * npos, 1.0)
    wh_loss = wh_sum / jnp.maximum(2.0 * npos, 1.0)
    pos_conf = pc_sum / jnp.maximum(npos, 1.0)
    neg_conf = nc_sum / jnp.maximum(float(_BOX_NUM * _N) - npos, 1.0)
    out_ref[0] = (class_loss + 2.0 * pos_conf + 0.5 * neg_conf
                  + 5.0 * xy_loss + 5.0 * wh_loss)
    out_ref[1] = class_loss
    out_ref[2] = xy_loss
    out_ref[3] = wh_loss
    out_ref[4] = pos_conf
    out_ref[5] = neg_conf


def _combine(sc_out, tc_out):
    return pl.pallas_call(
        _combine_body,
        in_specs=[
            pl.BlockSpec(memory_space=pltpu.VMEM),
            pl.BlockSpec(memory_space=pltpu.SMEM),
        ],
        out_specs=pl.BlockSpec(memory_space=pltpu.SMEM),
        out_shape=jax.ShapeDtypeStruct((6,), jnp.float32),
    )(sc_out, tc_out)


@jax.jit
def _yolo_loss(p, gt):
    pT = jnp.transpose(p, (1, 2, 0, 3))
    gtT = jnp.transpose(gt, (1, 2, 0, 3))
    loc, tc_out = _tc_class(pT, gtT)
    sc_out = _sc_loc(loc)
    out = _combine(sc_out, tc_out)
    return (out[0], out[1], out[2], out[3], out[4], out[5])


def kernel(p, gt):
    return _yolo_loss(p, gt)
